# Initial kernel scaffold; baseline (speedup 1.0000x reference)
import jax
import jax.numpy as jnp
from jax.experimental import pallas as pl


def kernel(target_bbox, rpn_match, rpn_bbox):
    raise NotImplementedError("write your pallas kernel here")


# speedup vs baseline: 7.7562x; 7.7562x over previous
import functools

import jax
import jax.numpy as jnp
from jax import lax
from jax.experimental import pallas as pl
from jax.experimental.pallas import tpu as pltpu
from jax.experimental.pallas import tpu_sc as plsc

NUM_IMAGES = 8
ANCHORS = 262144
MAX_TARGETS = 512
NW = 32
WPI = NW // NUM_IMAGES
CHUNK = ANCHORS // WPI
SEG = 8192
NSEG = CHUNK // SEG
L = 16
RPN_ROWS = NUM_IMAGES * ANCHORS * 4 // 128


def _sc_loss_kernel(match_hbm, tgt_hbm, rpn_hbm, out_sum, out_cnt,
                    seg_ref, idx_ref, tgt_ref, rows_ref, idx16_ref,
                    row16_ref, stage_f_ref, stage_i_ref, cnt_all_ref,
                    sem):
    c_idx = lax.axis_index("c")
    s_idx = lax.axis_index("s")
    wid = c_idx * 16 + s_idx
    image = wid // WPI
    quarter = s_idx % WPI
    qbase = (s_idx // WPI) * WPI
    chunk_start = wid * CHUNK

    iota16 = lax.iota(jnp.int32, L)
    zeros_i = jnp.zeros((L,), jnp.int32)
    zeros_f = jnp.zeros((L,), jnp.float32)

    pltpu.sync_copy(tgt_hbm.at[pl.ds(image * MAX_TARGETS * 4,
                                     MAX_TARGETS * 4)], tgt_ref)

    def seg_body(s, n_vec):
        pltpu.sync_copy(match_hbm.at[pl.ds(chunk_start + s * SEG, SEG)],
                        seg_ref)

        def vec_body(i, n_vec):
            v = seg_ref[pl.ds(i * L, L)]
            m = v == 1
            cum = jnp.cumsum(jnp.where(m, 1, 0))
            offs = n_vec + cum - 1
            ids = chunk_start + s * SEG + i * L + iota16
            plsc.store_scatter(idx_ref, [offs], ids, mask=m)
            return n_vec + plsc.all_reduce_population_count(m)

        return lax.fori_loop(0, SEG // L, vec_body, n_vec)

    n_vec = lax.fori_loop(0, NSEG, seg_body, zeros_i)

    plsc.store_scatter(idx_ref, [n_vec + iota16], zeros_i)

    stage_i_ref[...] = n_vec
    pltpu.sync_copy(stage_i_ref, out_cnt.at[wid])
    plsc.subcore_barrier()
    pltpu.sync_copy(out_cnt.at[pl.ds((wid // WPI) * WPI, WPI)], cnt_all_ref)
    base_vec = zeros_i
    qvec = jnp.full((L,), quarter, jnp.int32)
    for j in range(WPI):
        row = plsc.load_gather(cnt_all_ref,
                               [jnp.full((L,), j, jnp.int32), iota16])
        base_vec = base_vec + jnp.where(qvec > j, row, zeros_i)

    n_scalar = jnp.sum(jnp.where(iota16 == 0, n_vec, zeros_i))

    def pos_body(k, acc):
        idxv = idx_ref[pl.ds(k * L, L)]
        idx16_ref[...] = idxv
        row16_ref[...] = idxv >> 5
        pltpu.async_copy(rpn_hbm.at[row16_ref], rows_ref, sem).wait()
        for t in range(4):
            f = t * L + iota16
            r = f >> 2
            c = f & 3
            p = k * L + r
            rank = jnp.minimum(base_vec + p, MAX_TARGETS - 1)
            tgt = plsc.load_gather(tgt_ref, [rank * 4 + c])
            a = plsc.load_gather(idx16_ref, [r])
            col = ((a & 31) << 2) + c
            rpn = plsc.load_gather(rows_ref, [r, col])
            d = jnp.abs(tgt - rpn)
            sl1 = jnp.where(d < 1.0, 0.5 * d * d, d - 0.5)
            acc = acc + jnp.where(p < n_vec, sl1, zeros_f)
        return acc

    kmax = (n_scalar + (L - 1)) // L
    acc = lax.fori_loop(0, kmax, pos_body, zeros_f)

    stage_f_ref[...] = acc
    pltpu.sync_copy(stage_f_ref, out_sum.at[wid])


@jax.jit
def kernel(target_bbox, rpn_match, rpn_bbox):
    match_flat = jnp.reshape(rpn_match, (NUM_IMAGES * ANCHORS,))
    rpn_rows = jnp.reshape(rpn_bbox, (RPN_ROWS, 128))
    tgt_flat = jnp.reshape(target_bbox, (NUM_IMAGES * MAX_TARGETS * 4,))

    mesh = plsc.VectorSubcoreMesh(core_axis_name="c", subcore_axis_name="s")
    f = functools.partial(
        pl.kernel,
        mesh=mesh,
        compiler_params=pltpu.CompilerParams(needs_layout_passes=False),
        out_type=[
            jax.ShapeDtypeStruct((NW, L), jnp.float32),
            jax.ShapeDtypeStruct((NW, L), jnp.int32),
        ],
        scratch_types=[
            pltpu.VMEM((SEG,), jnp.int32),
            pltpu.VMEM((CHUNK + L,), jnp.int32),
            pltpu.VMEM((MAX_TARGETS * 4,), jnp.float32),
            pltpu.VMEM((L, 128), jnp.float32),
            pltpu.VMEM((L,), jnp.int32),
            pltpu.VMEM((L,), jnp.int32),
            pltpu.VMEM((L,), jnp.float32),
            pltpu.VMEM((L,), jnp.int32),
            pltpu.VMEM((WPI, L), jnp.int32),
            pltpu.SemaphoreType.DMA,
        ],
    )(_sc_loss_kernel)
    part_sum, part_cnt = f(match_flat, tgt_flat, rpn_rows)

    total = jnp.sum(part_sum)
    num_pos = jnp.sum(part_cnt[:, 0])
    num_elems = (num_pos * 4).astype(jnp.float32)
    return jnp.where(num_pos > 0, total / jnp.maximum(num_elems, 1.0),
                     jnp.asarray(0.0, dtype=jnp.float32))


# speedup vs baseline: 7.8367x; 1.0104x over previous
import functools

import jax
import jax.numpy as jnp
from jax import lax
from jax.experimental import pallas as pl
from jax.experimental.pallas import tpu as pltpu
from jax.experimental.pallas import tpu_sc as plsc

NUM_IMAGES = 8
ANCHORS = 262144
MAX_TARGETS = 512
NW = 32
WPI = NW // NUM_IMAGES
CHUNK = ANCHORS // WPI
SEG = 8192
NSEG = CHUNK // SEG
L = 16
RPN_ROWS = NUM_IMAGES * ANCHORS * 4 // 128


def _sc_loss_kernel(match_hbm, tgt_hbm, rpn_hbm, out_sum, out_cnt,
                    seg_ref, idx_ref, tgt_ref, rows_ref, idx16_ref,
                    row16_ref, stage_f_ref, stage_i_ref, cnt_all_ref,
                    sem):
    c_idx = lax.axis_index("c")
    s_idx = lax.axis_index("s")
    wid = c_idx * 16 + s_idx
    image = wid // WPI
    quarter = s_idx % WPI
    qbase = (s_idx // WPI) * WPI
    chunk_start = wid * CHUNK

    iota16 = lax.iota(jnp.int32, L)
    zeros_i = jnp.zeros((L,), jnp.int32)
    zeros_f = jnp.zeros((L,), jnp.float32)

    pltpu.sync_copy(tgt_hbm.at[pl.ds(image * MAX_TARGETS * 4,
                                     MAX_TARGETS * 4)], tgt_ref)

    KV = 16

    def seg_body(s, n_vec):
        pltpu.sync_copy(match_hbm.at[pl.ds(chunk_start + s * SEG, SEG)],
                        seg_ref)

        def blk_body(b, n_vec):
            base = b * (KV * L)
            acc = zeros_i
            for i in range(KV):
                v = seg_ref[pl.ds(base + i * L, L)]
                acc = acc + jnp.where(v == 1, 1, 0)
            tot = jnp.sum(acc)

            def scan_blk(n_vec):
                def vec_body(i, n_vec):
                    v = seg_ref[pl.ds(base + i * L, L)]
                    m = v == 1
                    cum = jnp.cumsum(jnp.where(m, 1, 0))
                    offs = n_vec + cum - 1
                    ids = chunk_start + s * SEG + base + i * L + iota16
                    plsc.store_scatter(idx_ref, [offs], ids, mask=m)
                    return n_vec + plsc.all_reduce_population_count(m)

                return lax.fori_loop(0, KV, vec_body, n_vec)

            return lax.cond(tot > 0, scan_blk, lambda n: n, n_vec)

        return lax.fori_loop(0, SEG // (KV * L), blk_body, n_vec)

    n_vec = lax.fori_loop(0, NSEG, seg_body, zeros_i)

    plsc.store_scatter(idx_ref, [n_vec + iota16], zeros_i)

    stage_i_ref[...] = n_vec
    pltpu.sync_copy(stage_i_ref, out_cnt.at[wid])
    plsc.subcore_barrier()
    pltpu.sync_copy(out_cnt.at[pl.ds((wid // WPI) * WPI, WPI)], cnt_all_ref)
    base_vec = zeros_i
    qvec = jnp.full((L,), quarter, jnp.int32)
    for j in range(WPI):
        row = plsc.load_gather(cnt_all_ref,
                               [jnp.full((L,), j, jnp.int32), iota16])
        base_vec = base_vec + jnp.where(qvec > j, row, zeros_i)

    n_scalar = jnp.sum(jnp.where(iota16 == 0, n_vec, zeros_i))

    def pos_body(k, acc):
        idxv = idx_ref[pl.ds(k * L, L)]
        idx16_ref[...] = idxv
        row16_ref[...] = idxv >> 5
        pltpu.async_copy(rpn_hbm.at[row16_ref], rows_ref, sem).wait()
        for t in range(4):
            f = t * L + iota16
            r = f >> 2
            c = f & 3
            p = k * L + r
            rank = jnp.minimum(base_vec + p, MAX_TARGETS - 1)
            tgt = plsc.load_gather(tgt_ref, [rank * 4 + c])
            a = plsc.load_gather(idx16_ref, [r])
            col = ((a & 31) << 2) + c
            rpn = plsc.load_gather(rows_ref, [r, col])
            d = jnp.abs(tgt - rpn)
            sl1 = jnp.where(d < 1.0, 0.5 * d * d, d - 0.5)
            acc = acc + jnp.where(p < n_vec, sl1, zeros_f)
        return acc

    kmax = (n_scalar + (L - 1)) // L
    acc = lax.fori_loop(0, kmax, pos_body, zeros_f)

    stage_f_ref[...] = acc
    pltpu.sync_copy(stage_f_ref, out_sum.at[wid])


@jax.jit
def kernel(target_bbox, rpn_match, rpn_bbox):
    match_flat = jnp.reshape(rpn_match, (NUM_IMAGES * ANCHORS,))
    rpn_rows = jnp.reshape(rpn_bbox, (RPN_ROWS, 128))
    tgt_flat = jnp.reshape(target_bbox, (NUM_IMAGES * MAX_TARGETS * 4,))

    mesh = plsc.VectorSubcoreMesh(core_axis_name="c", subcore_axis_name="s")
    f = functools.partial(
        pl.kernel,
        mesh=mesh,
        compiler_params=pltpu.CompilerParams(needs_layout_passes=False),
        out_type=[
            jax.ShapeDtypeStruct((NW, L), jnp.float32),
            jax.ShapeDtypeStruct((NW, L), jnp.int32),
        ],
        scratch_types=[
            pltpu.VMEM((SEG,), jnp.int32),
            pltpu.VMEM((CHUNK + L,), jnp.int32),
            pltpu.VMEM((MAX_TARGETS * 4,), jnp.float32),
            pltpu.VMEM((L, 128), jnp.float32),
            pltpu.VMEM((L,), jnp.int32),
            pltpu.VMEM((L,), jnp.int32),
            pltpu.VMEM((L,), jnp.float32),
            pltpu.VMEM((L,), jnp.int32),
            pltpu.VMEM((WPI, L), jnp.int32),
            pltpu.SemaphoreType.DMA,
        ],
    )(_sc_loss_kernel)
    part_sum, part_cnt = f(match_flat, tgt_flat, rpn_rows)

    total = jnp.sum(part_sum)
    num_pos = jnp.sum(part_cnt[:, 0])
    num_elems = (num_pos * 4).astype(jnp.float32)
    return jnp.where(num_pos > 0, total / jnp.maximum(num_elems, 1.0),
                     jnp.asarray(0.0, dtype=jnp.float32))


# speedup vs baseline: 7.8420x; 1.0007x over previous
import functools

import jax
import jax.numpy as jnp
from jax import lax
from jax.experimental import pallas as pl
from jax.experimental.pallas import tpu as pltpu
from jax.experimental.pallas import tpu_sc as plsc

NUM_IMAGES = 8
ANCHORS = 262144
MAX_TARGETS = 512
NW = 32
WPI = NW // NUM_IMAGES
CHUNK = ANCHORS // WPI
SEG = 8192
NSEG = CHUNK // SEG
L = 16


def _sc_loss_kernel(match_hbm, tgt_hbm, rpn_hbm, out_sum, out_cnt,
                    seg_ref, idx_ref, tgt_ref, rows_ref, idx16_ref,
                    stage_f_ref, stage_i_ref, cnt_all_ref,
                    sem):
    c_idx = lax.axis_index("c")
    s_idx = lax.axis_index("s")
    wid = c_idx * 16 + s_idx
    image = wid // WPI
    quarter = s_idx % WPI
    chunk_start = quarter * CHUNK
    gchunk_start = wid * CHUNK

    iota16 = lax.iota(jnp.int32, L)
    zeros_i = jnp.zeros((L,), jnp.int32)
    zeros_f = jnp.zeros((L,), jnp.float32)

    pltpu.sync_copy(tgt_hbm.at[pl.ds(image * MAX_TARGETS * 4,
                                     MAX_TARGETS * 4)], tgt_ref)

    KV = 16

    def seg_body(s, n_vec):
        pltpu.sync_copy(match_hbm.at[pl.ds(gchunk_start + s * SEG, SEG)],
                        seg_ref)

        def blk_body(b, n_vec):
            base = b * (KV * L)
            acc = zeros_i
            for i in range(KV):
                v = seg_ref[pl.ds(base + i * L, L)]
                acc = acc + jnp.where(v == 1, 1, 0)
            tot = jnp.sum(acc)

            def scan_blk(n_vec):
                def vec_body(i, n_vec):
                    v = seg_ref[pl.ds(base + i * L, L)]
                    m = v == 1
                    cum = jnp.cumsum(jnp.where(m, 1, 0))
                    offs = n_vec + cum - 1
                    ids = chunk_start + s * SEG + base + i * L + iota16
                    plsc.store_scatter(idx_ref, [offs], ids, mask=m)
                    return n_vec + plsc.all_reduce_population_count(m)

                return lax.fori_loop(0, KV, vec_body, n_vec)

            return lax.cond(tot > 0, scan_blk, lambda n: n, n_vec)

        return lax.fori_loop(0, SEG // (KV * L), blk_body, n_vec)

    n_vec = lax.fori_loop(0, NSEG, seg_body, zeros_i)

    plsc.store_scatter(idx_ref, [n_vec + iota16], zeros_i)

    stage_i_ref[...] = n_vec
    pltpu.sync_copy(stage_i_ref, out_cnt.at[wid])
    plsc.subcore_barrier()
    pltpu.sync_copy(out_cnt.at[pl.ds((wid // WPI) * WPI, WPI)], cnt_all_ref)
    base_vec = zeros_i
    qvec = jnp.full((L,), quarter, jnp.int32)
    for j in range(WPI):
        row = plsc.load_gather(cnt_all_ref,
                               [jnp.full((L,), j, jnp.int32), iota16])
        base_vec = base_vec + jnp.where(qvec > j, row, zeros_i)

    n_scalar = jnp.sum(jnp.where(iota16 == 0, n_vec, zeros_i))

    img_base = image * (ANCHORS * 4)

    def pos_body(k, acc):
        idxv = idx_ref[pl.ds(k * L, L)]
        idx16_ref[...] = idxv
        copies = []
        for j in range(L):
            a_j = jnp.sum(jnp.where(iota16 == j, idxv, zeros_i))
            off = pl.multiple_of(img_base + (a_j & ~31) * 4, 128)
            copies.append(
                pltpu.async_copy(rpn_hbm.at[pl.ds(off, 128)],
                                 rows_ref.at[pl.ds(j * 128, 128)], sem))
        for cp in copies:
            cp.wait()
        for t in range(4):
            f = t * L + iota16
            r = f >> 2
            c = f & 3
            p = k * L + r
            rank = jnp.minimum(base_vec + p, MAX_TARGETS - 1)
            tgt = plsc.load_gather(tgt_ref, [rank * 4 + c])
            a = plsc.load_gather(idx16_ref, [r])
            rpn = plsc.load_gather(rows_ref, [r * 128 + (a & 31) * 4 + c])
            d = jnp.abs(tgt - rpn)
            sl1 = jnp.where(d < 1.0, 0.5 * d * d, d - 0.5)
            acc = acc + jnp.where(p < n_vec, sl1, zeros_f)
        return acc

    kmax = (n_scalar + (L - 1)) // L
    acc = lax.fori_loop(0, kmax, pos_body, zeros_f)

    stage_f_ref[...] = acc
    pltpu.sync_copy(stage_f_ref, out_sum.at[wid])


@jax.jit
def kernel(target_bbox, rpn_match, rpn_bbox):
    match_flat = jnp.reshape(rpn_match, (NUM_IMAGES * ANCHORS,))
    tgt_flat = jnp.reshape(target_bbox, (NUM_IMAGES * MAX_TARGETS * 4,))
    rpn_flat = jnp.reshape(rpn_bbox, (NUM_IMAGES * ANCHORS * 4,))

    mesh = plsc.VectorSubcoreMesh(core_axis_name="c", subcore_axis_name="s")
    f = functools.partial(
        pl.kernel,
        mesh=mesh,
        compiler_params=pltpu.CompilerParams(needs_layout_passes=False),
        out_type=[
            jax.ShapeDtypeStruct((NW, L), jnp.float32),
            jax.ShapeDtypeStruct((NW, L), jnp.int32),
        ],
        scratch_types=[
            pltpu.VMEM((SEG,), jnp.int32),
            pltpu.VMEM((CHUNK + L,), jnp.int32),
            pltpu.VMEM((MAX_TARGETS * 4,), jnp.float32),
            pltpu.VMEM((L * 128,), jnp.float32),
            pltpu.VMEM((L,), jnp.int32),
            pltpu.VMEM((L,), jnp.float32),
            pltpu.VMEM((L,), jnp.int32),
            pltpu.VMEM((WPI, L), jnp.int32),
            pltpu.SemaphoreType.DMA,
        ],
    )(_sc_loss_kernel)
    part_sum, part_cnt = f(match_flat, tgt_flat, rpn_flat)

    total = jnp.sum(part_sum)
    num_pos = jnp.sum(part_cnt[:, 0])
    num_elems = (num_pos * 4).astype(jnp.float32)
    return jnp.where(num_pos > 0, total / jnp.maximum(num_elems, 1.0),
                     jnp.asarray(0.0, dtype=jnp.float32))
